# baseline (speedup 1.0000x reference)
import functools

import jax
import jax.numpy as jnp
from jax import lax
from jax.experimental import pallas as pl
from jax.experimental.pallas import tpu as pltpu
from jax.experimental.pallas import tpu_sc as plsc

_START = -4.0
_END = 4.0
_GRID = 1000
_TBL = 1024

_A = (_GRID - 1) / (_END - _START)
_B = -_START * _A

_NC = 2
_NS = 16
_L = 16

_RCH = 32


def _table_body(bg_ref, bgs_ref, cp_ref, ga_ref, gb_ref):
    cp = cp_ref[...]
    ga = jnp.sum(bg_ref[...] * cp, axis=1, keepdims=True)
    gs = jnp.sum(bgs_ref[...] * cp, axis=1, keepdims=True)
    ga_ref[...] = ga
    gb_ref[...] = gs - ga


def _fold_tables(basis_grid, control_points):
    pad = _TBL - _GRID
    bg = jnp.pad(basis_grid, ((0, pad), (0, 0)))
    bgs = jnp.pad(basis_grid[1:], ((0, pad + 1), (0, 0)))
    cp = control_points.reshape(1, -1)
    ga, gb = pl.pallas_call(
        _table_body,
        out_shape=(
            jax.ShapeDtypeStruct((_TBL, 1), jnp.float32),
            jax.ShapeDtypeStruct((_TBL, 1), jnp.float32),
        ),
    )(bg, bgs, cp)
    return ga.reshape(_TBL), gb.reshape(_TBL)


def _make_lookup(shape):
    nd, nrows, d = shape
    assert nd == _NC and nrows % _NS == 0 and d % _L == 0
    rows_w = nrows // _NS
    n_ch = rows_w // _RCH
    assert rows_w % _RCH == 0 and n_ch >= 2
    vpr = d // _L
    mesh = plsc.VectorSubcoreMesh(
        core_axis_name="c", subcore_axis_name="s",
        num_cores=_NC, num_subcores=_NS,
    )

    @functools.partial(
        pl.kernel,
        mesh=mesh,
        out_type=jax.ShapeDtypeStruct(shape, jnp.float32),
        scratch_types=[
            pltpu.VMEM((_TBL,), jnp.float32),
            pltpu.VMEM((_TBL,), jnp.float32),
            pltpu.VMEM((_RCH, d), jnp.float32),
            pltpu.VMEM((_RCH, d), jnp.float32),
            pltpu.VMEM((_RCH, d), jnp.float32),
            pltpu.VMEM((_RCH, d), jnp.float32),
            pltpu.SemaphoreType.DMA,
            pltpu.SemaphoreType.DMA,
            pltpu.SemaphoreType.DMA,
            pltpu.SemaphoreType.DMA,
        ],
        compiler_params=pltpu.CompilerParams(
            needs_layout_passes=False, use_tc_tiling_on_sc=False,
        ),
    )
    def lookup(x_hbm, ga_hbm, gb_hbm, out_hbm,
               ga_v, gb_v, xa, xb, oa, ob, sxa, sxb, soa, sob):
        c = lax.axis_index("c")
        s = lax.axis_index("s")
        r0 = s * rows_w
        pltpu.sync_copy(ga_hbm, ga_v)
        pltpu.sync_copy(gb_hbm, gb_v)
        xbuf, obuf = [xa, xb], [oa, ob]
        xsem, osem = [sxa, sxb], [soa, sob]

        def start_in(k):
            return pltpu.async_copy(
                x_hbm.at[c, pl.ds(r0 + k * _RCH, _RCH), :],
                xbuf[k % 2], xsem[k % 2],
            )

        in_copies = {0: start_in(0)}
        out_copies = {}
        for k in range(n_ch):
            if k + 1 < n_ch:
                in_copies[k + 1] = start_in(k + 1)
            in_copies[k].wait()
            if k >= 2:
                out_copies[k - 2].wait()
            xv_ref, ov_ref = xbuf[k % 2], obuf[k % 2]

            def row_body(r, carry, xv_ref=xv_ref, ov_ref=ov_ref):
                for u in range(vpr):
                    o = u * _L
                    xv = xv_ref[r, pl.ds(o, _L)]
                    s = xv * _A
                    s = jnp.minimum(jnp.maximum(s, -_B), _B)
                    t = s + _B
                    ii = t.astype(jnp.int32)
                    w = t - ii.astype(jnp.float32)
                    a = plsc.load_gather(ga_v, [ii])
                    b = plsc.load_gather(gb_v, [ii])
                    ov_ref[r, pl.ds(o, _L)] = a + w * b
                return carry

            lax.fori_loop(0, _RCH, row_body, 0)
            out_copies[k] = pltpu.async_copy(
                obuf[k % 2],
                out_hbm.at[c, pl.ds(r0 + k * _RCH, _RCH), :],
                osem[k % 2],
            )
        out_copies[n_ch - 2].wait()
        out_copies[n_ch - 1].wait()

    return lookup


def kernel(x, control_points, basis_grid):
    ga, gb = _fold_tables(basis_grid, control_points)
    return _make_lookup(x.shape)(x, ga, gb)


# speedup vs baseline: 1.3531x; 1.3531x over previous
import functools

import jax
import jax.numpy as jnp
from jax import lax
from jax.experimental import pallas as pl
from jax.experimental.pallas import tpu as pltpu
from jax.experimental.pallas import tpu_sc as plsc

_START = -4.0
_END = 4.0
_GRID = 1000
_TBL = 1024

_A = (_GRID - 1) / (_END - _START)
_B = -_START * _A

_NC = 2
_NS = 16
_L = 16

_RCH = 32


def _table_body(bg_ref, cp_ref, ga_ref, gb_ref):
    ga = jnp.sum(bg_ref[...] * cp_ref[...], axis=1, keepdims=True)
    ga_ref[...] = ga
    gb_ref[...] = jnp.roll(ga, -1, axis=0) - ga


def _fold_tables(basis_grid, control_points):
    bg = jnp.pad(basis_grid, ((0, _TBL - _GRID), (0, 0)))
    cp = control_points.reshape(1, -1)
    ga, gb = pl.pallas_call(
        _table_body,
        out_shape=(
            jax.ShapeDtypeStruct((_TBL, 1), jnp.float32),
            jax.ShapeDtypeStruct((_TBL, 1), jnp.float32),
        ),
    )(bg, cp)
    return ga.reshape(_TBL), gb.reshape(_TBL)


def _make_lookup(shape):
    nd, nrows, d = shape
    assert nd == _NC and nrows % _NS == 0 and d % _L == 0
    rows_w = nrows // _NS
    n_ch = rows_w // _RCH
    assert rows_w % _RCH == 0 and n_ch >= 2
    vpr = d // _L
    mesh = plsc.VectorSubcoreMesh(
        core_axis_name="c", subcore_axis_name="s",
        num_cores=_NC, num_subcores=_NS,
    )

    @functools.partial(
        pl.kernel,
        mesh=mesh,
        out_type=jax.ShapeDtypeStruct(shape, jnp.float32),
        scratch_types=[
            pltpu.VMEM((_TBL,), jnp.float32),
            pltpu.VMEM((_TBL,), jnp.float32),
            pltpu.VMEM((_RCH, d), jnp.float32),
            pltpu.VMEM((_RCH, d), jnp.float32),
            pltpu.VMEM((_RCH, d), jnp.float32),
            pltpu.VMEM((_RCH, d), jnp.float32),
            pltpu.SemaphoreType.DMA,
            pltpu.SemaphoreType.DMA,
            pltpu.SemaphoreType.DMA,
            pltpu.SemaphoreType.DMA,
        ],
        compiler_params=pltpu.CompilerParams(needs_layout_passes=False),
    )
    def lookup(x_hbm, ga_hbm, gb_hbm, out_hbm,
               ga_v, gb_v, xa, xb, oa, ob, sxa, sxb, soa, sob):
        c = lax.axis_index("c")
        s = lax.axis_index("s")
        r0 = s * rows_w
        pltpu.sync_copy(ga_hbm, ga_v)
        pltpu.sync_copy(gb_hbm, gb_v)
        xbuf, obuf = [xa, xb], [oa, ob]
        xsem, osem = [sxa, sxb], [soa, sob]

        def start_in(k):
            return pltpu.async_copy(
                x_hbm.at[c, pl.ds(r0 + k * _RCH, _RCH), :],
                xbuf[k % 2], xsem[k % 2],
            )

        in_copies = {0: start_in(0)}
        out_copies = {}
        for k in range(n_ch):
            if k + 1 < n_ch:
                in_copies[k + 1] = start_in(k + 1)
            in_copies[k].wait()
            if k >= 2:
                out_copies[k - 2].wait()
            xv_ref, ov_ref = xbuf[k % 2], obuf[k % 2]

            def row_body(r, carry, xv_ref=xv_ref, ov_ref=ov_ref):
                for u in range(vpr):
                    o = u * _L
                    xv = xv_ref[r, pl.ds(o, _L)]
                    s = xv * _A
                    s = jnp.minimum(jnp.maximum(s, -_B), _B)
                    t = s + _B
                    ii = t.astype(jnp.int32)
                    w = t - ii.astype(jnp.float32)
                    a = plsc.load_gather(ga_v, [ii])
                    b = plsc.load_gather(gb_v, [ii])
                    ov_ref[r, pl.ds(o, _L)] = a + w * b
                return carry

            lax.fori_loop(0, _RCH, row_body, 0)
            out_copies[k] = pltpu.async_copy(
                obuf[k % 2],
                out_hbm.at[c, pl.ds(r0 + k * _RCH, _RCH), :],
                osem[k % 2],
            )
        out_copies[n_ch - 2].wait()
        out_copies[n_ch - 1].wait()

    return lookup


def kernel(x, control_points, basis_grid):
    ga, gb = _fold_tables(basis_grid, control_points)
    return _make_lookup(x.shape)(x, ga, gb)
